# baseline (speedup 1.0000x reference)
import functools

import jax
import jax.numpy as jnp
from jax.experimental import pallas as pl

N = 10000
E = 320000
KS = (2000, 1000, 500)
F32 = jnp.float32


def _dot(a, b):
    return jnp.dot(a, b, preferred_element_type=F32)


def _xw_scale_body(x_ref, w_ref, indeg_ref, t_ref):
    dis = jax.lax.rsqrt(indeg_ref[...] + 2.0)
    t_ref[...] = _dot(x_ref[...], w_ref[...]) * dis


def _lvl0_post_body(m_ref, t_ref, indeg_ref, b_ref, pw_ref, x_ref, s_ref):
    dis = jax.lax.rsqrt(indeg_ref[...] + 2.0)
    h = (m_ref[...] + 2.0 * t_ref[...]) * dis + b_ref[...][None, :]
    x0 = jnp.maximum(h, 0.0)
    x_ref[...] = x0
    pw = pw_ref[...]
    inv_norm = jax.lax.rsqrt(jnp.sum(pw * pw))
    s_ref[...] = jnp.tanh(_dot(x0, pw[:, None]) * inv_norm)


def _lvl0_final_body(m_ref, t_ref, indeg_ref, b_ref, o_ref):
    dis = jax.lax.rsqrt(indeg_ref[...] + 2.0)
    h = (m_ref[...] + 2.0 * t_ref[...]) * dis + b_ref[...][None, :]
    mx = jnp.max(h, axis=1, keepdims=True)
    lse = mx + jnp.log(jnp.sum(jnp.exp(h - mx), axis=1, keepdims=True))
    o_ref[...] = h - lse


def _gcn_dense_score_body(a_ref, x_ref, w_ref, b_ref, pw_ref, o_ref, s_ref):
    A = a_ref[...]
    deg = jnp.sum(A, axis=1, keepdims=True) + 2.0
    dis = jax.lax.rsqrt(deg)
    yn = _dot(x_ref[...], w_ref[...]) * dis
    h = (_dot(A, yn) + 2.0 * yn) * dis + b_ref[...][None, :]
    xo = jnp.maximum(h, 0.0)
    o_ref[...] = xo
    pw = pw_ref[...]
    inv_norm = jax.lax.rsqrt(jnp.sum(pw * pw))
    s_ref[...] = jnp.tanh(_dot(xo, pw[:, None]) * inv_norm)


def _gcn_dense_body(a_ref, x_ref, w_ref, b_ref, o_ref, *, relu):
    A = a_ref[...]
    deg = jnp.sum(A, axis=1, keepdims=True) + 2.0
    dis = jax.lax.rsqrt(deg)
    yn = _dot(x_ref[...], w_ref[...]) * dis
    h = (_dot(A, yn) + 2.0 * yn) * dis + b_ref[...][None, :]
    o_ref[...] = jnp.maximum(h, 0.0) if relu else h


def _augmm_body(r_ref, c_ref, o_ref, *, mb):
    m = pl.program_id(0)
    k = pl.program_id(1)

    @pl.when(k == 0)
    def _():
        o_ref[...] = jnp.zeros_like(o_ref)

    o_ref[...] += _dot(r_ref[...], c_ref[...])

    @pl.when(k == pl.num_programs(1) - 1)
    def _():
        bm, n = o_ref.shape
        ii = jax.lax.broadcasted_iota(jnp.int32, (bm, n), 0) + m * mb
        jj = jax.lax.broadcasted_iota(jnp.int32, (bm, n), 1)
        o_ref[...] = jnp.where(ii == jj, 0.0, o_ref[...])


def _xw_scale(x, w, indeg):
    n = x.shape[0]
    return pl.pallas_call(
        _xw_scale_body,
        out_shape=jax.ShapeDtypeStruct((n, w.shape[1]), F32),
    )(x, w, indeg)


def _lvl0_post(m, t, indeg, b, pw):
    n = m.shape[0]
    return pl.pallas_call(
        _lvl0_post_body,
        out_shape=(
            jax.ShapeDtypeStruct((n, m.shape[1]), F32),
            jax.ShapeDtypeStruct((n, 1), F32),
        ),
    )(m, t, indeg, b, pw)


def _lvl0_final(m, t, indeg, b):
    n = m.shape[0]
    return pl.pallas_call(
        _lvl0_final_body,
        out_shape=jax.ShapeDtypeStruct((n, m.shape[1]), F32),
    )(m, t, indeg, b)


def _gcn_dense_score(A, x, w, b, pw):
    n = A.shape[0]
    return pl.pallas_call(
        _gcn_dense_score_body,
        out_shape=(
            jax.ShapeDtypeStruct((n, w.shape[1]), F32),
            jax.ShapeDtypeStruct((n, 1), F32),
        ),
    )(A, x, w, b, pw)


def _gcn_dense(A, x, w, b, relu):
    n = A.shape[0]
    return pl.pallas_call(
        functools.partial(_gcn_dense_body, relu=relu),
        out_shape=jax.ShapeDtypeStruct((n, w.shape[1]), F32),
    )(A, x, w, b)


def _aug_matmul(R, C, mb, bk):
    k_new, kdim = R.shape
    msteps = k_new // mb
    ksteps = kdim // bk
    return pl.pallas_call(
        functools.partial(_augmm_body, mb=mb),
        grid=(msteps, ksteps),
        in_specs=[
            pl.BlockSpec((mb, bk), lambda m, k: (m, k)),
            pl.BlockSpec((bk, k_new), lambda m, k: (k, 0)),
        ],
        out_specs=pl.BlockSpec((mb, k_new), lambda m, k: (m, 0)),
        out_shape=jax.ShapeDtypeStruct((k_new, k_new), F32),
    )(R, C)


def _scatter_rows_add(n, vals, idx):
    return jnp.zeros((n, vals.shape[1]), F32).at[idx].add(vals)


def kernel(x, edge_index, W0, b0, W1, b1, W2, b2, W3, b3,
           pw1, pw2, pw3, U0, ub0, U1, ub1, U2, ub2):
    src = edge_index[0]
    dst = edge_index[1]
    ones_e = jnp.ones((E,), F32)

    indeg = jnp.zeros((N,), F32).at[dst].add(ones_e)[:, None]

    t0 = _xw_scale(x, W0, indeg)
    m0 = _scatter_rows_add(N, t0[src], dst)
    x0, s1 = _lvl0_post(m0, t0, indeg, b0, pw1)
    s1 = s1[:, 0]

    vals1, perm1 = jax.lax.top_k(s1, KS[0])
    k1 = KS[0]
    arange1 = jnp.arange(k1, dtype=jnp.int32)
    inv1 = jnp.full((N,), k1, jnp.int32).at[perm1].set(arange1)

    kpad = ((N + 2047) // 2048) * 2048
    rows = (
        jnp.zeros((k1 + 8, kpad), F32)
        .at[inv1[dst], src].add(ones_e, mode="drop")
        .at[arange1, perm1].add(1.0)
    )[:k1]
    cols = (
        jnp.zeros((kpad, k1 + 8), F32)
        .at[dst, inv1[src]].add(ones_e, mode="drop")
        .at[perm1, arange1].add(1.0)
    )[:, :k1]
    A1 = _aug_matmul(rows, cols, mb=k1 // 2, bk=512)

    x1p = x0[perm1] * vals1[:, None]
    x1, s2 = _gcn_dense_score(A1, x1p, W1, b1, pw2)
    s2 = s2[:, 0]

    k2 = KS[1]
    vals2, perm2 = jax.lax.top_k(s2, k2)
    arange2 = jnp.arange(k2, dtype=jnp.int32)
    rows2 = A1[perm2].at[arange2, perm2].add(1.0)
    cols2 = A1[:, perm2].at[perm2, arange2].add(1.0)
    A2 = _aug_matmul(rows2, cols2, mb=k2, bk=k1)
    x2p = x1[perm2] * vals2[:, None]
    x2, s3 = _gcn_dense_score(A2, x2p, W2, b2, pw3)
    s3 = s3[:, 0]

    k3 = KS[2]
    vals3, perm3 = jax.lax.top_k(s3, k3)
    arange3 = jnp.arange(k3, dtype=jnp.int32)
    rows3 = A2[perm3].at[arange3, perm3].add(1.0)
    cols3 = A2[:, perm3].at[perm3, arange3].add(1.0)
    A3 = _aug_matmul(rows3, cols3, mb=k3, bk=k2)
    x3p = x2[perm3] * vals3[:, None]
    x3 = _gcn_dense(A3, x3p, W3, b3, relu=True)

    xu2 = x2.at[perm3].add(x3)
    h2 = _gcn_dense(A2, xu2, U0, ub0, relu=True)
    xu1 = x1.at[perm2].add(h2)
    h1 = _gcn_dense(A1, xu1, U1, ub1, relu=True)
    xu0 = x0.at[perm1].add(h1)

    tf = _xw_scale(xu0, U2, indeg)
    mf = _scatter_rows_add(N, tf[src], dst)
    return _lvl0_final(mf, tf, indeg, ub2)


# speedup vs baseline: 1.0194x; 1.0194x over previous
import functools

import jax
import jax.numpy as jnp
from jax.experimental import pallas as pl

N = 10000
E = 320000
KS = (2000, 1000, 500)
F32 = jnp.float32


def _dot(a, b):
    return jnp.dot(a, b, preferred_element_type=F32)


def _xw_scale_body(x_ref, w_ref, indeg_ref, t_ref):
    dis = jax.lax.rsqrt(indeg_ref[...] + 2.0)
    t_ref[...] = _dot(x_ref[...], w_ref[...]) * dis


def _lvl0_post_body(m_ref, t_ref, indeg_ref, b_ref, pw_ref, x_ref, s_ref):
    dis = jax.lax.rsqrt(indeg_ref[...] + 2.0)
    h = (m_ref[...] + 2.0 * t_ref[...]) * dis + b_ref[...][None, :]
    x0 = jnp.maximum(h, 0.0)
    x_ref[...] = x0
    pw = pw_ref[...]
    inv_norm = jax.lax.rsqrt(jnp.sum(pw * pw))
    s_ref[...] = jnp.tanh(_dot(x0, pw[:, None]) * inv_norm)


def _lvl0_final_body(m_ref, t_ref, indeg_ref, b_ref, o_ref):
    dis = jax.lax.rsqrt(indeg_ref[...] + 2.0)
    h = (m_ref[...] + 2.0 * t_ref[...]) * dis + b_ref[...][None, :]
    mx = jnp.max(h, axis=1, keepdims=True)
    lse = mx + jnp.log(jnp.sum(jnp.exp(h - mx), axis=1, keepdims=True))
    o_ref[...] = h - lse


def _gcn_dense_score_body(a_ref, x_ref, w_ref, b_ref, pw_ref, o_ref, s_ref):
    A = a_ref[...]
    deg = jnp.sum(A, axis=1, keepdims=True) + 2.0
    dis = jax.lax.rsqrt(deg)
    yn = _dot(x_ref[...], w_ref[...]) * dis
    h = (_dot(A, yn) + 2.0 * yn) * dis + b_ref[...][None, :]
    xo = jnp.maximum(h, 0.0)
    o_ref[...] = xo
    pw = pw_ref[...]
    inv_norm = jax.lax.rsqrt(jnp.sum(pw * pw))
    s_ref[...] = jnp.tanh(_dot(xo, pw[:, None]) * inv_norm)


def _gcn_dense_body(a_ref, x_ref, w_ref, b_ref, o_ref, *, relu):
    A = a_ref[...]
    deg = jnp.sum(A, axis=1, keepdims=True) + 2.0
    dis = jax.lax.rsqrt(deg)
    yn = _dot(x_ref[...], w_ref[...]) * dis
    h = (_dot(A, yn) + 2.0 * yn) * dis + b_ref[...][None, :]
    o_ref[...] = jnp.maximum(h, 0.0) if relu else h


def _augmm_body(r_ref, c_ref, o_ref, *, mb):
    m = pl.program_id(0)
    k = pl.program_id(1)

    @pl.when(k == 0)
    def _():
        o_ref[...] = jnp.zeros_like(o_ref)

    o_ref[...] += _dot(r_ref[...], c_ref[...])

    @pl.when(k == pl.num_programs(1) - 1)
    def _():
        bm, n = o_ref.shape
        ii = jax.lax.broadcasted_iota(jnp.int32, (bm, n), 0) + m * mb
        jj = jax.lax.broadcasted_iota(jnp.int32, (bm, n), 1)
        o_ref[...] = jnp.where(ii == jj, 0.0, o_ref[...])


def _xw_scale(x, w, indeg):
    n = x.shape[0]
    return pl.pallas_call(
        _xw_scale_body,
        out_shape=jax.ShapeDtypeStruct((n, w.shape[1]), F32),
    )(x, w, indeg)


def _lvl0_post(m, t, indeg, b, pw):
    n = m.shape[0]
    return pl.pallas_call(
        _lvl0_post_body,
        out_shape=(
            jax.ShapeDtypeStruct((n, m.shape[1]), F32),
            jax.ShapeDtypeStruct((n, 1), F32),
        ),
    )(m, t, indeg, b, pw)


def _lvl0_final(m, t, indeg, b):
    n = m.shape[0]
    return pl.pallas_call(
        _lvl0_final_body,
        out_shape=jax.ShapeDtypeStruct((n, m.shape[1]), F32),
    )(m, t, indeg, b)


def _gcn_dense_score(A, x, w, b, pw):
    n = A.shape[0]
    return pl.pallas_call(
        _gcn_dense_score_body,
        out_shape=(
            jax.ShapeDtypeStruct((n, w.shape[1]), F32),
            jax.ShapeDtypeStruct((n, 1), F32),
        ),
    )(A, x, w, b, pw)


def _gcn_dense(A, x, w, b, relu):
    n = A.shape[0]
    return pl.pallas_call(
        functools.partial(_gcn_dense_body, relu=relu),
        out_shape=jax.ShapeDtypeStruct((n, w.shape[1]), F32),
    )(A, x, w, b)


def _aug_matmul(R, C, mb, bk):
    k_new, kdim = R.shape
    msteps = k_new // mb
    ksteps = kdim // bk
    return pl.pallas_call(
        functools.partial(_augmm_body, mb=mb),
        grid=(msteps, ksteps),
        in_specs=[
            pl.BlockSpec((mb, bk), lambda m, k: (m, k)),
            pl.BlockSpec((bk, k_new), lambda m, k: (k, 0)),
        ],
        out_specs=pl.BlockSpec((mb, k_new), lambda m, k: (m, 0)),
        out_shape=jax.ShapeDtypeStruct((k_new, k_new), F32),
    )(R, C)


def _scatter_rows_add(n, vals, idx):
    return jnp.zeros((n, vals.shape[1]), F32).at[idx].add(vals)


def kernel(x, edge_index, W0, b0, W1, b1, W2, b2, W3, b3,
           pw1, pw2, pw3, U0, ub0, U1, ub1, U2, ub2):
    src = edge_index[0]
    dst = edge_index[1]
    ones_e = jnp.ones((E,), F32)

    indeg = jnp.zeros((N,), F32).at[dst].add(ones_e)[:, None]

    t0 = _xw_scale(x, W0, indeg)
    m0 = _scatter_rows_add(N, t0[src], dst)
    x0, s1 = _lvl0_post(m0, t0, indeg, b0, pw1)
    s1 = s1[:, 0]

    vals1, perm1 = jax.lax.top_k(s1, KS[0])
    k1 = KS[0]
    arange1 = jnp.arange(k1, dtype=jnp.int32)
    inv1 = jnp.full((N,), k1, jnp.int32).at[perm1].set(arange1)

    kpad = ((N + 2047) // 2048) * 2048
    src_ext = jnp.concatenate([src, perm1])
    dst_ext = jnp.concatenate([dst, perm1])
    ones_ext = jnp.ones((E + k1,), F32)
    rows = jnp.zeros((k1, kpad), F32).at[inv1[dst_ext], src_ext].add(
        ones_ext, mode="drop")
    cols = jnp.zeros((kpad, k1), F32).at[dst_ext, inv1[src_ext]].add(
        ones_ext, mode="drop")
    A1 = _aug_matmul(rows, cols, mb=k1 // 2, bk=512)

    x1p = x0[perm1] * vals1[:, None]
    x1, s2 = _gcn_dense_score(A1, x1p, W1, b1, pw2)
    s2 = s2[:, 0]

    k2 = KS[1]
    vals2, perm2 = jax.lax.top_k(s2, k2)
    arange2 = jnp.arange(k2, dtype=jnp.int32)
    rows2 = A1[perm2].at[arange2, perm2].add(1.0)
    cols2 = A1[:, perm2].at[perm2, arange2].add(1.0)
    A2 = _aug_matmul(rows2, cols2, mb=k2, bk=k1)
    x2p = x1[perm2] * vals2[:, None]
    x2, s3 = _gcn_dense_score(A2, x2p, W2, b2, pw3)
    s3 = s3[:, 0]

    k3 = KS[2]
    vals3, perm3 = jax.lax.top_k(s3, k3)
    arange3 = jnp.arange(k3, dtype=jnp.int32)
    rows3 = A2[perm3].at[arange3, perm3].add(1.0)
    cols3 = A2[:, perm3].at[perm3, arange3].add(1.0)
    A3 = _aug_matmul(rows3, cols3, mb=k3, bk=k2)
    x3p = x2[perm3] * vals3[:, None]
    x3 = _gcn_dense(A3, x3p, W3, b3, relu=True)

    xu2 = x2.at[perm3].add(x3)
    h2 = _gcn_dense(A2, xu2, U0, ub0, relu=True)
    xu1 = x1.at[perm2].add(h2)
    h1 = _gcn_dense(A1, xu1, U1, ub1, relu=True)
    xu0 = x0.at[perm1].add(h1)

    tf = _xw_scale(xu0, U2, indeg)
    mf = _scatter_rows_add(N, tf[src], dst)
    return _lvl0_final(mf, tf, indeg, ub2)


# speedup vs baseline: 1.4590x; 1.4313x over previous
import functools

import jax
import jax.numpy as jnp
from jax import lax
from jax.experimental import pallas as pl
from jax.experimental.pallas import tpu as pltpu
from jax.experimental.pallas import tpu_sc as plsc

N = 10000
E = 320000
KS = (2000, 1000, 500)
F32 = jnp.float32

NC, NS = 2, 16
NW = NC * NS
EPT = E // NW
CH = 80
NCHUNK = EPT // CH


def _dot(a, b):
    return jnp.dot(a, b, preferred_element_type=F32)


def _sc_mesh():
    return plsc.VectorSubcoreMesh(core_axis_name="c", subcore_axis_name="s")


def _sc_indeg(dst, ones_rows, zeros_acc):
    @functools.partial(
        pl.kernel,
        mesh=_sc_mesh(),
        compiler_params=pltpu.CompilerParams(use_tc_tiling_on_sc=False),
        out_type=jax.ShapeDtypeStruct((NC, N, 8), F32),
        scratch_types=[
            pltpu.VMEM((CH,), jnp.int32),
            pltpu.VMEM((CH, 8), F32),
            pltpu.VMEM_SHARED((N, 8), F32),
        ],
    )
    def k(dst_hbm, ones_hbm, zero_hbm, out_hbm, dst_v, ones_v, acc_sh):
        cid = lax.axis_index("c")
        sid = lax.axis_index("s")

        @pl.when(sid == 0)
        def _():
            pltpu.sync_copy(zero_hbm, acc_sh)

        pltpu.sync_copy(ones_hbm, ones_v)
        plsc.subcore_barrier()
        base = (sid * NC + cid) * EPT

        def body(j, carry):
            pltpu.sync_copy(dst_hbm.at[pl.ds(base + j * CH, CH)], dst_v)
            pltpu.sync_copy(ones_v, acc_sh.at[dst_v], add=True)
            return carry

        lax.fori_loop(0, NCHUNK, body, 0)
        plsc.subcore_barrier()

        @pl.when(sid == 0)
        def _():
            pltpu.sync_copy(acc_sh, out_hbm.at[cid])

    return k(dst, ones_rows, zeros_acc)


def _sc_msg(t, src, dst, zeros_acc):
    @functools.partial(
        pl.kernel,
        mesh=_sc_mesh(),
        compiler_params=pltpu.CompilerParams(use_tc_tiling_on_sc=False),
        out_type=jax.ShapeDtypeStruct((NC, N, 64), F32),
        scratch_types=[
            pltpu.VMEM((CH,), jnp.int32),
            pltpu.VMEM((CH,), jnp.int32),
            pltpu.VMEM((CH, 64), F32),
            pltpu.VMEM_SHARED((N, 64), F32),
            pltpu.SemaphoreType.DMA,
        ],
    )
    def k(t_hbm, src_hbm, dst_hbm, zero_hbm, out_hbm,
          src_v, dst_v, rows_v, acc_sh, sem):
        cid = lax.axis_index("c")
        sid = lax.axis_index("s")

        @pl.when(sid == 0)
        def _():
            pltpu.sync_copy(zero_hbm, acc_sh)

        plsc.subcore_barrier()
        base = (sid * NC + cid) * EPT

        def body(j, carry):
            off = base + j * CH
            pltpu.sync_copy(src_hbm.at[pl.ds(off, CH)], src_v)
            pltpu.sync_copy(dst_hbm.at[pl.ds(off, CH)], dst_v)
            pltpu.async_copy(t_hbm.at[src_v], rows_v, sem).wait()
            pltpu.sync_copy(rows_v, acc_sh.at[dst_v], add=True)
            return carry

        lax.fori_loop(0, NCHUNK, body, 0)
        plsc.subcore_barrier()

        @pl.when(sid == 0)
        def _():
            pltpu.sync_copy(acc_sh, out_hbm.at[cid])

    return k(t, src, dst, zeros_acc)


def _indeg_of(ind_ref):
    return ind_ref[0, :, 0:1] + ind_ref[1, :, 0:1]


def _xw_scale_body(x_ref, w_ref, ind_ref, t_ref):
    dis = jax.lax.rsqrt(_indeg_of(ind_ref) + 2.0)
    t_ref[...] = _dot(x_ref[...], w_ref[...]) * dis


def _lvl0_post_body(m_ref, t_ref, ind_ref, b_ref, pw_ref, x_ref, s_ref):
    dis = jax.lax.rsqrt(_indeg_of(ind_ref) + 2.0)
    m = m_ref[0] + m_ref[1]
    h = (m + 2.0 * t_ref[...]) * dis + b_ref[...][None, :]
    x0 = jnp.maximum(h, 0.0)
    x_ref[...] = x0
    pw = pw_ref[...]
    inv_norm = jax.lax.rsqrt(jnp.sum(pw * pw))
    s_ref[...] = jnp.tanh(_dot(x0, pw[:, None]) * inv_norm)


def _lvl0_final_body(m_ref, t_ref, ind_ref, b_ref, o_ref):
    dis = jax.lax.rsqrt(_indeg_of(ind_ref) + 2.0)
    m = m_ref[0] + m_ref[1]
    h = (m + 2.0 * t_ref[...]) * dis + b_ref[...][None, :]
    mx = jnp.max(h, axis=1, keepdims=True)
    lse = mx + jnp.log(jnp.sum(jnp.exp(h - mx), axis=1, keepdims=True))
    o_ref[...] = h - lse


def _gcn_dense_score_body(a_ref, x_ref, w_ref, b_ref, pw_ref, o_ref, s_ref):
    A = a_ref[...]
    deg = jnp.sum(A, axis=1, keepdims=True) + 2.0
    dis = jax.lax.rsqrt(deg)
    yn = _dot(x_ref[...], w_ref[...]) * dis
    h = (_dot(A, yn) + 2.0 * yn) * dis + b_ref[...][None, :]
    xo = jnp.maximum(h, 0.0)
    o_ref[...] = xo
    pw = pw_ref[...]
    inv_norm = jax.lax.rsqrt(jnp.sum(pw * pw))
    s_ref[...] = jnp.tanh(_dot(xo, pw[:, None]) * inv_norm)


def _gcn_dense_body(a_ref, x_ref, w_ref, b_ref, o_ref, *, relu):
    A = a_ref[...]
    deg = jnp.sum(A, axis=1, keepdims=True) + 2.0
    dis = jax.lax.rsqrt(deg)
    yn = _dot(x_ref[...], w_ref[...]) * dis
    h = (_dot(A, yn) + 2.0 * yn) * dis + b_ref[...][None, :]
    o_ref[...] = jnp.maximum(h, 0.0) if relu else h


def _augmm_body(r_ref, c_ref, o_ref, *, mb):
    m = pl.program_id(0)
    k = pl.program_id(1)

    @pl.when(k == 0)
    def _():
        o_ref[...] = jnp.zeros_like(o_ref)

    o_ref[...] += _dot(r_ref[...], c_ref[...])

    @pl.when(k == pl.num_programs(1) - 1)
    def _():
        bm, n = o_ref.shape
        ii = jax.lax.broadcasted_iota(jnp.int32, (bm, n), 0) + m * mb
        jj = jax.lax.broadcasted_iota(jnp.int32, (bm, n), 1)
        o_ref[...] = jnp.where(ii == jj, 0.0, o_ref[...])


def _augmm_t_body(r_ref, ct_ref, o_ref):
    out = jax.lax.dot_general(
        r_ref[...], ct_ref[...], (((1,), (1,)), ((), ())),
        preferred_element_type=F32)
    n = out.shape[0]
    ii = jax.lax.broadcasted_iota(jnp.int32, (n, n), 0)
    jj = jax.lax.broadcasted_iota(jnp.int32, (n, n), 1)
    o_ref[...] = jnp.where(ii == jj, 0.0, out)


def _transpose_body(a_ref, o_ref):
    o_ref[...] = a_ref[...].T


def _xw_scale(x, w, ind):
    n = x.shape[0]
    return pl.pallas_call(
        _xw_scale_body,
        out_shape=jax.ShapeDtypeStruct((n, w.shape[1]), F32),
    )(x, w, ind)


def _lvl0_post(m, t, ind, b, pw):
    n = t.shape[0]
    return pl.pallas_call(
        _lvl0_post_body,
        out_shape=(
            jax.ShapeDtypeStruct((n, t.shape[1]), F32),
            jax.ShapeDtypeStruct((n, 1), F32),
        ),
    )(m, t, ind, b, pw)


def _lvl0_final(m, t, ind, b):
    n = t.shape[0]
    return pl.pallas_call(
        _lvl0_final_body,
        out_shape=jax.ShapeDtypeStruct((n, t.shape[1]), F32),
    )(m, t, ind, b)


def _gcn_dense_score(A, x, w, b, pw):
    n = A.shape[0]
    return pl.pallas_call(
        _gcn_dense_score_body,
        out_shape=(
            jax.ShapeDtypeStruct((n, w.shape[1]), F32),
            jax.ShapeDtypeStruct((n, 1), F32),
        ),
    )(A, x, w, b, pw)


def _gcn_dense(A, x, w, b, relu):
    n = A.shape[0]
    return pl.pallas_call(
        functools.partial(_gcn_dense_body, relu=relu),
        out_shape=jax.ShapeDtypeStruct((n, w.shape[1]), F32),
    )(A, x, w, b)


def _aug_matmul(R, C, mb, bk):
    k_new, kdim = R.shape
    return pl.pallas_call(
        functools.partial(_augmm_body, mb=mb),
        grid=(k_new // mb, kdim // bk),
        in_specs=[
            pl.BlockSpec((mb, bk), lambda m, k: (m, k)),
            pl.BlockSpec((bk, k_new), lambda m, k: (k, 0)),
        ],
        out_specs=pl.BlockSpec((mb, k_new), lambda m, k: (m, 0)),
        out_shape=jax.ShapeDtypeStruct((k_new, k_new), F32),
    )(R, C)


def _aug_matmul_t(R, Ct):
    k_new = R.shape[0]
    return pl.pallas_call(
        _augmm_t_body,
        out_shape=jax.ShapeDtypeStruct((k_new, k_new), F32),
    )(R, Ct)


def _transpose(A):
    n = A.shape[0]
    return pl.pallas_call(
        _transpose_body,
        out_shape=jax.ShapeDtypeStruct((n, n), F32),
    )(A)


def kernel(x, edge_index, W0, b0, W1, b1, W2, b2, W3, b3,
           pw1, pw2, pw3, U0, ub0, U1, ub1, U2, ub2):
    src = jnp.asarray(edge_index[0], jnp.int32)
    dst = jnp.asarray(edge_index[1], jnp.int32)

    ones_rows = jnp.ones((CH, 8), F32)
    zeros8 = jnp.zeros((N, 8), F32)
    zeros64 = jnp.zeros((N, 64), F32)

    ind = _sc_indeg(dst, ones_rows, zeros8)

    t0 = _xw_scale(x, W0, ind)
    m0 = _sc_msg(t0, src, dst, zeros64)
    x0, s1 = _lvl0_post(m0, t0, ind, b0, pw1)
    s1 = s1[:, 0]

    k1 = KS[0]
    vals1, perm1 = jax.lax.top_k(s1, k1)
    arange1 = jnp.arange(k1, dtype=jnp.int32)
    inv1 = jnp.full((N,), k1, jnp.int32).at[perm1].set(arange1)

    kpad = ((N + 2047) // 2048) * 2048
    src_ext = jnp.concatenate([src, perm1])
    dst_ext = jnp.concatenate([dst, perm1])
    ones_ext = jnp.ones((E + k1,), F32)
    rows = jnp.zeros((k1, kpad), F32).at[inv1[dst_ext], src_ext].add(
        ones_ext, mode="drop")
    cols = jnp.zeros((kpad, k1), F32).at[dst_ext, inv1[src_ext]].add(
        ones_ext, mode="drop")
    A1 = _aug_matmul(rows, cols, mb=k1 // 2, bk=512)

    x1p = x0[perm1] * vals1[:, None]
    x1, s2 = _gcn_dense_score(A1, x1p, W1, b1, pw2)
    s2 = s2[:, 0]

    k2 = KS[1]
    vals2, perm2 = jax.lax.top_k(s2, k2)
    arange2 = jnp.arange(k2, dtype=jnp.int32)
    A1t = _transpose(A1)
    rows2 = A1[perm2].at[arange2, perm2].add(1.0)
    cols2t = A1t[perm2].at[arange2, perm2].add(1.0)
    A2 = _aug_matmul_t(rows2, cols2t)
    x2p = x1[perm2] * vals2[:, None]
    x2, s3 = _gcn_dense_score(A2, x2p, W2, b2, pw3)
    s3 = s3[:, 0]

    k3 = KS[2]
    vals3, perm3 = jax.lax.top_k(s3, k3)
    arange3 = jnp.arange(k3, dtype=jnp.int32)
    A2t = _transpose(A2)
    rows3 = A2[perm3].at[arange3, perm3].add(1.0)
    cols3t = A2t[perm3].at[arange3, perm3].add(1.0)
    A3 = _aug_matmul_t(rows3, cols3t)
    x3p = x2[perm3] * vals3[:, None]
    x3 = _gcn_dense(A3, x3p, W3, b3, relu=True)

    xu2 = x2.at[perm3].add(x3)
    h2 = _gcn_dense(A2, xu2, U0, ub0, relu=True)
    xu1 = x1.at[perm2].add(h2)
    h1 = _gcn_dense(A1, xu1, U1, ub1, relu=True)
    xu0 = x0.at[perm1].add(h1)

    tf = _xw_scale(xu0, U2, ind)
    mf = _sc_msg(tf, src, dst, zeros64)
    return _lvl0_final(mf, tf, ind, ub2)
